# baseline (speedup 1.0000x reference)
import functools

import jax
import jax.numpy as jnp
from jax import lax
from jax.experimental import pallas as pl
from jax.experimental.pallas import tpu as pltpu
from jax.experimental.pallas import tpu_sc as plsc

_N_ACTIONS = 1000
_BATCH = 16384
_NW = 32
_RPW = _BATCH // _NW
_NBLK = _BATCH // 8
_BPW = _NBLK // _NW
_CB = 2
_NCH = _BPW // _CB
_L = 16

_mesh = plsc.VectorSubcoreMesh(core_axis_name="c", subcore_axis_name="s")


@functools.partial(
    pl.kernel,
    out_type=jax.ShapeDtypeStruct((_BATCH,), jnp.float32),
    mesh=_mesh,
    scratch_types=[
        pltpu.VMEM((_RPW,), jnp.int32),
        *[pltpu.VMEM((_CB, 8, _N_ACTIONS), jnp.float32) for _ in range(4)],
        pltpu.VMEM((_RPW,), jnp.float32),
        *[pltpu.SemaphoreType.DMA for _ in range(4)],
    ],
    compiler_params=pltpu.CompilerParams(needs_layout_passes=False),
)
def _gather_kernel(mat_hbm, idx_hbm, out_hbm,
                   idx_v, buf_a, buf_b, buf_c, buf_d, out_v,
                   sem_a, sem_b, sem_c, sem_d):
    wid = lax.axis_index("s") * 2 + lax.axis_index("c")
    base = wid * _RPW
    blk0 = wid * _BPW

    pltpu.sync_copy(idx_hbm.at[pl.ds(base, _RPW)], idx_v)

    bufs = (buf_a, buf_b, buf_c, buf_d)
    sems = (sem_a, sem_b, sem_c, sem_d)
    copies = [None, None, None, None]
    rpc = _CB * 8

    def start(c):
        b = c % 4
        copies[b] = pltpu.async_copy(
            mat_hbm.at[pl.ds(blk0 + c * _CB, _CB)], bufs[b], sems[b])

    def extract(c):
        b = c % 4
        copies[b].wait()
        buf = bufs[b]
        for s in range(rpc // _L):
            off = c * rpc + s * _L
            cols = idx_v[pl.ds(off, _L)]
            local = s * _L + lax.iota(jnp.int32, _L)
            b16 = lax.shift_right_logical(local, 3)
            r16 = local & 7
            out_v[pl.ds(off, _L)] = plsc.load_gather(buf, [b16, r16, cols])

    for c in range(3):
        start(c)
    for c in range(_NCH):
        if c + 3 < _NCH:
            start(c + 3)
        extract(c)

    pltpu.sync_copy(out_v, out_hbm.at[pl.ds(base, _RPW)])


def kernel(full_output, indices):
    mat = full_output.reshape(_NBLK, 8, _N_ACTIONS)
    idx = indices.astype(jnp.int32)
    return _gather_kernel(mat, idx)
